# baseline (speedup 1.0000x reference)
import functools

import jax
import jax.numpy as jnp
from jax.experimental import pallas as pl
from jax.experimental.pallas import tpu as pltpu

_EPS = 1e-5
_VMEM_LIMIT = 64 * 1024 * 1024


def _cparams():
    return pltpu.CompilerParams(
        dimension_semantics=("parallel",),
        vmem_limit_bytes=_VMEM_LIMIT,
    )


def _whole(shape):
    shape = tuple(shape)
    return pl.BlockSpec(shape, lambda n: (0,) * len(shape))


def _im2col(xp, H, W):
    C = xp.shape[-1]
    return jnp.concatenate(
        [xp[dy:dy + H, dx:dx + W, :].reshape(H * W, C)
         for dy in range(3) for dx in range(3)], axis=-1)


def _scale_shift_rows(sum_row, sq_row, count):
    mean = sum_row / count
    var = jnp.maximum(sq_row / count - mean * mean, 0.0)
    inv = jax.lax.rsqrt(var + _EPS)
    return inv, -mean * inv


def _conv1_body(x_ref, w1_ref, b1_ref, y1_ref, st_ref, g_ref, xp,
                *, H, W, imgs):
    cin = x_ref.shape[-1]
    cout = w1_ref.shape[-1]

    s1 = jnp.zeros((1, cout), jnp.float32)
    q1 = jnp.zeros((1, cout), jnp.float32)
    xs = jnp.zeros((1, cin), jnp.float32)
    gacc = jnp.zeros((cin, cin), jnp.float32)
    xp[...] = jnp.zeros((H + 2, W + 2, cin), jnp.bfloat16)
    for i in range(imgs):
        xb = x_ref[i].astype(jnp.bfloat16)
        xp[1:1 + H, 1:1 + W, :] = xb.reshape(H, W, cin)

        patches = _im2col(xp[...], H, W)
        y1 = jnp.dot(patches, w1_ref[...],
                     preferred_element_type=jnp.float32) + b1_ref[...]
        s1 = s1 + jnp.sum(y1, axis=0, keepdims=True)
        q1 = q1 + jnp.sum(y1 * y1, axis=0, keepdims=True)
        y1_ref[i] = y1.astype(jnp.bfloat16)

        xs = xs + jnp.sum(x_ref[i], axis=0, keepdims=True)
        gacc = gacc + jax.lax.dot_general(
            xb, xb, (((0,), (0,)), ((), ())),
            preferred_element_type=jnp.float32)

    if cout > cin:
        xs = jnp.concatenate(
            [xs, jnp.zeros((1, cout - cin), jnp.float32)], axis=1)
    else:
        xs = xs[:, :cout]
    st_ref[...] = jnp.concatenate([s1, q1, xs], axis=0)[None]
    g_ref[...] = gacc[None]


def _conv2_body(y1_ref, sta_ref, w2_ref, b2_ref, y2_ref, st_ref, hp,
                *, H, W, imgs, count):
    cout = w2_ref.shape[-1]
    tot = jnp.sum(sta_ref[...], axis=0)
    scale, shift = _scale_shift_rows(tot[0:1], tot[1:2], count)

    s2 = jnp.zeros((1, cout), jnp.float32)
    q2 = jnp.zeros((1, cout), jnp.float32)
    hp[...] = jnp.zeros((H + 2, W + 2, cout), jnp.bfloat16)
    for i in range(imgs):
        h1 = jnp.maximum(y1_ref[i].astype(jnp.float32) * scale + shift, 0.0)
        hp[1:1 + H, 1:1 + W, :] = h1.astype(jnp.bfloat16).reshape(H, W, cout)

        patches = _im2col(hp[...], H, W)
        y2 = jnp.dot(patches, w2_ref[...],
                     preferred_element_type=jnp.float32) + b2_ref[...]
        s2 = s2 + jnp.sum(y2, axis=0, keepdims=True)
        q2 = q2 + jnp.sum(y2 * y2, axis=0, keepdims=True)
        y2_ref[i] = y2.astype(jnp.bfloat16)

    st_ref[...] = jnp.concatenate([s2, q2], axis=0)[None]


def _add_relu_body(y2_ref, x_ref, ws_ref, stb_ref, bn_ref, o_ref,
                   *, imgs, count):
    totb = jnp.sum(stb_ref[...], axis=0)
    s2, t2 = _scale_shift_rows(totb[0:1], totb[1:2], count)
    bn = bn_ref[...]
    ss = bn[0:1]
    ts = bn[1:2]
    bs = bn[2:3]
    for i in range(imgs):
        sc = jnp.dot(x_ref[i].astype(jnp.bfloat16), ws_ref[...],
                     preferred_element_type=jnp.float32) + bs
        y2 = y2_ref[i].astype(jnp.float32) * s2 + t2
        o_ref[i] = jnp.maximum(y2 + sc * ss + ts, 0.0)


def kernel(x, w1, b1, w2, b2, ws, bs):
    N, cin, H, W = x.shape
    cout = w1.shape[-1]
    HW = H * W
    count = N * HW
    imgs = 16 if N % 16 == 0 else (8 if N % 8 == 0 else
                                   (2 if N % 2 == 0 else 1))
    imgs_c = 4 if N % 4 == 0 else 1
    G = N // imgs
    Gc = N // imgs_c

    xh = jnp.transpose(x, (0, 2, 3, 1)).astype(jnp.float32).reshape(N, HW, cin)

    w1f = w1.reshape(9 * cin, cout).astype(jnp.bfloat16)
    w2f = w2.reshape(9 * cout, cout).astype(jnp.bfloat16)
    wsf = ws.astype(jnp.bfloat16)
    b1f = b1.reshape(1, cout)
    b2f = b2.reshape(1, cout)

    x_spec = pl.BlockSpec((imgs, HW, cin), lambda n: (n, 0, 0))
    xc_spec = pl.BlockSpec((imgs_c, HW, cin), lambda n: (n, 0, 0))
    row_spec = pl.BlockSpec((imgs, HW, cout), lambda n: (n, 0, 0))
    rowc_spec = pl.BlockSpec((imgs_c, HW, cout), lambda n: (n, 0, 0))
    st3_spec = pl.BlockSpec((1, 3, cout), lambda n: (n, 0, 0))
    st2_spec = pl.BlockSpec((1, 2, cout), lambda n: (n, 0, 0))
    g_spec = pl.BlockSpec((1, cin, cin), lambda n: (n, 0, 0))

    y1, sta, gram = pl.pallas_call(
        functools.partial(_conv1_body, H=H, W=W, imgs=imgs),
        grid=(G,),
        in_specs=[x_spec, _whole(w1f.shape), _whole((1, cout))],
        out_specs=(row_spec, st3_spec, g_spec),
        out_shape=(
            jax.ShapeDtypeStruct((N, HW, cout), jnp.bfloat16),
            jax.ShapeDtypeStruct((G, 3, cout), jnp.float32),
            jax.ShapeDtypeStruct((G, cin, cin), jnp.float32),
        ),
        scratch_shapes=[pltpu.VMEM((H + 2, W + 2, cin), jnp.bfloat16)],
        compiler_params=_cparams(),
        cost_estimate=pl.CostEstimate(
            flops=2 * count * (9 * cin + cin) * cout, transcendentals=0,
            bytes_accessed=4 * count * cin + 2 * count * cout),
    )(xh, w1f, b1f)

    y2, stb = pl.pallas_call(
        functools.partial(_conv2_body, H=H, W=W, imgs=imgs, count=count),
        grid=(G,),
        in_specs=[row_spec, _whole((G, 3, cout)), _whole(w2f.shape),
                  _whole((1, cout))],
        out_specs=(row_spec, st2_spec),
        out_shape=(jax.ShapeDtypeStruct((N, HW, cout), jnp.bfloat16),
                   jax.ShapeDtypeStruct((G, 2, cout), jnp.float32)),
        scratch_shapes=[pltpu.VMEM((H + 2, W + 2, cout), jnp.bfloat16)],
        compiler_params=_cparams(),
        cost_estimate=pl.CostEstimate(
            flops=2 * count * 9 * cout * cout, transcendentals=0,
            bytes_accessed=2 * 2 * count * cout),
    )(y1, sta, w2f, b2f)

    tota = jnp.sum(sta, axis=0)
    xs = tota[2][:cin]
    Gm = jnp.sum(gram, axis=0)
    wsf32 = wsf.astype(jnp.float32)
    mproj = xs @ wsf32
    ssum = mproj + count * bs
    qsum = (jnp.sum(wsf32 * (Gm @ wsf32), axis=0)
            + 2.0 * bs * mproj + count * bs * bs)
    ss_, ts_ = _scale_shift_rows(ssum[None], qsum[None], count)
    bnrows = jnp.concatenate([ss_, ts_, bs[None]], axis=0)

    out = pl.pallas_call(
        functools.partial(_add_relu_body, imgs=imgs_c, count=count),
        grid=(Gc,),
        in_specs=[rowc_spec, xc_spec, _whole(wsf.shape),
                  _whole((G, 2, cout)), _whole((3, cout))],
        out_specs=rowc_spec,
        out_shape=jax.ShapeDtypeStruct((N, HW, cout), jnp.float32),
        compiler_params=_cparams(),
        cost_estimate=pl.CostEstimate(
            flops=2 * count * cin * cout + 6 * count * cout, transcendentals=0,
            bytes_accessed=2 * count * cout + 4 * count * cin
                           + 4 * count * cout),
    )(y2, xh, wsf, stb, bnrows)

    return jnp.transpose(out.reshape(N, H, W, cout), (0, 3, 1, 2))


# speedup vs baseline: 1.0159x; 1.0159x over previous
import functools

import jax
import jax.numpy as jnp
from jax.experimental import pallas as pl
from jax.experimental.pallas import tpu as pltpu

_EPS = 1e-5
_VMEM_LIMIT = 64 * 1024 * 1024


def _cparams():
    return pltpu.CompilerParams(
        dimension_semantics=("parallel",),
        vmem_limit_bytes=_VMEM_LIMIT,
    )


def _whole(shape):
    shape = tuple(shape)
    return pl.BlockSpec(shape, lambda n: (0,) * len(shape))


def _im2col(xp, H, W):
    C = xp.shape[-1]
    return jnp.concatenate(
        [xp[dy:dy + H, dx:dx + W, :].reshape(H * W, C)
         for dy in range(3) for dx in range(3)], axis=-1)


def _scale_shift_rows(sum_row, sq_row, count):
    mean = sum_row / count
    var = jnp.maximum(sq_row / count - mean * mean, 0.0)
    inv = jax.lax.rsqrt(var + _EPS)
    return inv, -mean * inv


def _conv1_body(x_ref, w1_ref, b1_ref, y1_ref, st_ref, g_ref, xp,
                *, H, W, imgs):
    cin = x_ref.shape[-1]
    cout = w1_ref.shape[-1]

    s1 = jnp.zeros((1, cout), jnp.float32)
    q1 = jnp.zeros((1, cout), jnp.float32)
    xs = jnp.zeros((1, cin), jnp.float32)
    gacc = jnp.zeros((cin, cin), jnp.float32)
    xp[...] = jnp.zeros((H + 2, W + 2, cin), jnp.bfloat16)
    for i in range(imgs):
        xb = x_ref[i].astype(jnp.bfloat16)
        xp[1:1 + H, 1:1 + W, :] = xb.reshape(H, W, cin)

        patches = _im2col(xp[...], H, W)
        y1 = jnp.dot(patches, w1_ref[...],
                     preferred_element_type=jnp.float32) + b1_ref[...]
        s1 = s1 + jnp.sum(y1, axis=0, keepdims=True)
        q1 = q1 + jnp.sum(y1 * y1, axis=0, keepdims=True)
        y1_ref[i] = y1.astype(jnp.bfloat16)

        xs = xs + jnp.sum(x_ref[i], axis=0, keepdims=True)
        gacc = gacc + jax.lax.dot_general(
            xb, xb, (((0,), (0,)), ((), ())),
            preferred_element_type=jnp.float32)

    if cout > cin:
        xs = jnp.concatenate(
            [xs, jnp.zeros((1, cout - cin), jnp.float32)], axis=1)
    else:
        xs = xs[:, :cout]
    st_ref[...] = jnp.concatenate([s1, q1, xs], axis=0)[None]
    g_ref[...] = gacc[None]


def _conv2_body(y1_ref, sta_ref, w2_ref, b2_ref, y2_ref, st_ref, hp,
                *, H, W, imgs, count):
    cout = w2_ref.shape[-1]
    tot = jnp.sum(sta_ref[...], axis=0)
    scale, shift = _scale_shift_rows(tot[0:1], tot[1:2], count)

    s2 = jnp.zeros((1, cout), jnp.float32)
    q2 = jnp.zeros((1, cout), jnp.float32)
    hp[...] = jnp.zeros((H + 2, W + 2, cout), jnp.bfloat16)
    for i in range(imgs):
        h1 = jnp.maximum(y1_ref[i].astype(jnp.float32) * scale + shift, 0.0)
        hp[1:1 + H, 1:1 + W, :] = h1.astype(jnp.bfloat16).reshape(H, W, cout)

        patches = _im2col(hp[...], H, W)
        y2 = jnp.dot(patches, w2_ref[...],
                     preferred_element_type=jnp.float32) + b2_ref[...]
        s2 = s2 + jnp.sum(y2, axis=0, keepdims=True)
        q2 = q2 + jnp.sum(y2 * y2, axis=0, keepdims=True)
        y2_ref[i] = y2.astype(jnp.bfloat16)

    st_ref[...] = jnp.concatenate([s2, q2], axis=0)[None]


def _add_relu_body(y2_ref, x_ref, ws_ref, stb_ref, bn_ref, o_ref,
                   *, imgs, count):
    totb = jnp.sum(stb_ref[...], axis=0)
    s2, t2 = _scale_shift_rows(totb[0:1], totb[1:2], count)
    bn = bn_ref[...]
    ss = bn[0:1]
    ts = bn[1:2]
    bs = bn[2:3]
    for i in range(imgs):
        sc = jnp.dot(x_ref[i].astype(jnp.bfloat16), ws_ref[...],
                     preferred_element_type=jnp.float32) + bs
        y2 = y2_ref[i].astype(jnp.float32) * s2 + t2
        o_ref[i] = jnp.maximum(y2 + sc * ss + ts, 0.0)


def kernel(x, w1, b1, w2, b2, ws, bs):
    N, cin, H, W = x.shape
    cout = w1.shape[-1]
    HW = H * W
    count = N * HW
    imgs = 8 if N % 8 == 0 else (2 if N % 2 == 0 else 1)
    imgs_c = 4 if N % 4 == 0 else 1
    G = N // imgs
    Gc = N // imgs_c

    xh = jnp.transpose(x, (0, 2, 3, 1)).astype(jnp.float32).reshape(N, HW, cin)

    w1f = w1.reshape(9 * cin, cout).astype(jnp.bfloat16)
    w2f = w2.reshape(9 * cout, cout).astype(jnp.bfloat16)
    wsf = ws.astype(jnp.bfloat16)
    b1f = b1.reshape(1, cout)
    b2f = b2.reshape(1, cout)

    x_spec = pl.BlockSpec((imgs, HW, cin), lambda n: (n, 0, 0))
    xc_spec = pl.BlockSpec((imgs_c, HW, cin), lambda n: (n, 0, 0))
    row_spec = pl.BlockSpec((imgs, HW, cout), lambda n: (n, 0, 0))
    rowc_spec = pl.BlockSpec((imgs_c, HW, cout), lambda n: (n, 0, 0))
    st3_spec = pl.BlockSpec((1, 3, cout), lambda n: (n, 0, 0))
    st2_spec = pl.BlockSpec((1, 2, cout), lambda n: (n, 0, 0))
    g_spec = pl.BlockSpec((1, cin, cin), lambda n: (n, 0, 0))

    y1, sta, gram = pl.pallas_call(
        functools.partial(_conv1_body, H=H, W=W, imgs=imgs),
        grid=(G,),
        in_specs=[x_spec, _whole(w1f.shape), _whole((1, cout))],
        out_specs=(row_spec, st3_spec, g_spec),
        out_shape=(
            jax.ShapeDtypeStruct((N, HW, cout), jnp.bfloat16),
            jax.ShapeDtypeStruct((G, 3, cout), jnp.float32),
            jax.ShapeDtypeStruct((G, cin, cin), jnp.float32),
        ),
        scratch_shapes=[pltpu.VMEM((H + 2, W + 2, cin), jnp.bfloat16)],
        compiler_params=_cparams(),
        cost_estimate=pl.CostEstimate(
            flops=2 * count * (9 * cin + cin) * cout, transcendentals=0,
            bytes_accessed=4 * count * cin + 2 * count * cout),
    )(xh, w1f, b1f)

    y2, stb = pl.pallas_call(
        functools.partial(_conv2_body, H=H, W=W, imgs=imgs, count=count),
        grid=(G,),
        in_specs=[row_spec, _whole((G, 3, cout)), _whole(w2f.shape),
                  _whole((1, cout))],
        out_specs=(row_spec, st2_spec),
        out_shape=(jax.ShapeDtypeStruct((N, HW, cout), jnp.bfloat16),
                   jax.ShapeDtypeStruct((G, 2, cout), jnp.float32)),
        scratch_shapes=[pltpu.VMEM((H + 2, W + 2, cout), jnp.bfloat16)],
        compiler_params=_cparams(),
        cost_estimate=pl.CostEstimate(
            flops=2 * count * 9 * cout * cout, transcendentals=0,
            bytes_accessed=2 * 2 * count * cout),
    )(y1, sta, w2f, b2f)

    tota = jnp.sum(sta, axis=0)
    xs = tota[2][:cin]
    Gm = jnp.sum(gram, axis=0)
    wsf32 = wsf.astype(jnp.float32)
    mproj = xs @ wsf32
    ssum = mproj + count * bs
    qsum = (jnp.sum(wsf32 * (Gm @ wsf32), axis=0)
            + 2.0 * bs * mproj + count * bs * bs)
    ss_, ts_ = _scale_shift_rows(ssum[None], qsum[None], count)
    bnrows = jnp.concatenate([ss_, ts_, bs[None]], axis=0)

    out = pl.pallas_call(
        functools.partial(_add_relu_body, imgs=imgs_c, count=count),
        grid=(Gc,),
        in_specs=[rowc_spec, xc_spec, _whole(wsf.shape),
                  _whole((G, 2, cout)), _whole((3, cout))],
        out_specs=rowc_spec,
        out_shape=jax.ShapeDtypeStruct((N, HW, cout), jnp.float32),
        compiler_params=_cparams(),
        cost_estimate=pl.CostEstimate(
            flops=2 * count * cin * cout + 6 * count * cout, transcendentals=0,
            bytes_accessed=2 * count * cout + 4 * count * cin
                           + 4 * count * cout),
    )(y2, xh, wsf, stb, bnrows)

    return jnp.transpose(out.reshape(N, H, W, cout), (0, 3, 1, 2))


# speedup vs baseline: 1.0239x; 1.0078x over previous
import functools

import jax
import jax.numpy as jnp
from jax.experimental import pallas as pl
from jax.experimental.pallas import tpu as pltpu

_EPS = 1e-5
_VMEM_LIMIT = 64 * 1024 * 1024


def _cparams():
    return pltpu.CompilerParams(
        dimension_semantics=("parallel",),
        vmem_limit_bytes=_VMEM_LIMIT,
    )


def _whole(shape):
    shape = tuple(shape)
    return pl.BlockSpec(shape, lambda n: (0,) * len(shape))


def _im2col(xp, H, W):
    C = xp.shape[-1]
    return jnp.concatenate(
        [xp[dy:dy + H, dx:dx + W, :].reshape(H * W, C)
         for dy in range(3) for dx in range(3)], axis=-1)


def _scale_shift_rows(sum_row, sq_row, count):
    mean = sum_row / count
    var = jnp.maximum(sq_row / count - mean * mean, 0.0)
    inv = jax.lax.rsqrt(var + _EPS)
    return inv, -mean * inv


def _conv1_body(x_ref, w1_ref, b1_ref, y1_ref, st_ref, g_ref, xp,
                *, H, W, imgs):
    cin = x_ref.shape[-1]
    cout = w1_ref.shape[-1]

    s1 = jnp.zeros((1, cout), jnp.float32)
    q1 = jnp.zeros((1, cout), jnp.float32)
    xs = jnp.zeros((1, cin), jnp.float32)
    gacc = jnp.zeros((cin, cin), jnp.float32)
    xp[...] = jnp.zeros((H + 2, W + 2, cin), jnp.bfloat16)
    for i in range(imgs):
        xb = x_ref[i].astype(jnp.bfloat16)
        xp[1:1 + H, 1:1 + W, :] = xb.reshape(H, W, cin)

        patches = _im2col(xp[...], H, W)
        y1 = jnp.dot(patches, w1_ref[...],
                     preferred_element_type=jnp.float32) + b1_ref[...]
        s1 = s1 + jnp.sum(y1, axis=0, keepdims=True)
        q1 = q1 + jnp.sum(y1 * y1, axis=0, keepdims=True)
        y1_ref[i] = y1.astype(jnp.bfloat16)

        xs = xs + jnp.sum(x_ref[i], axis=0, keepdims=True)
        gacc = gacc + jax.lax.dot_general(
            xb, xb, (((0,), (0,)), ((), ())),
            preferred_element_type=jnp.float32)

    if cout > cin:
        xs = jnp.concatenate(
            [xs, jnp.zeros((1, cout - cin), jnp.float32)], axis=1)
    else:
        xs = xs[:, :cout]
    st_ref[...] = jnp.concatenate([s1, q1, xs], axis=0)[None]
    g_ref[...] = gacc[None]


def _conv2_body(y1_ref, sta_ref, w2_ref, b2_ref, y2_ref, st_ref, hp,
                *, H, W, imgs, count):
    cout = w2_ref.shape[-1]
    tot = jnp.sum(sta_ref[...], axis=0)
    scale, shift = _scale_shift_rows(tot[0:1], tot[1:2], count)

    s2 = jnp.zeros((1, cout), jnp.float32)
    q2 = jnp.zeros((1, cout), jnp.float32)
    hp[...] = jnp.zeros((H + 2, W + 2, cout), jnp.bfloat16)
    for i in range(imgs):
        h1 = jnp.maximum(y1_ref[i].astype(jnp.float32) * scale + shift, 0.0)
        hp[1:1 + H, 1:1 + W, :] = h1.astype(jnp.bfloat16).reshape(H, W, cout)

        patches = _im2col(hp[...], H, W)
        y2 = jnp.dot(patches, w2_ref[...],
                     preferred_element_type=jnp.float32) + b2_ref[...]
        s2 = s2 + jnp.sum(y2, axis=0, keepdims=True)
        q2 = q2 + jnp.sum(y2 * y2, axis=0, keepdims=True)
        y2_ref[i] = y2.astype(jnp.bfloat16)

    st_ref[...] = jnp.concatenate([s2, q2], axis=0)[None]


def _add_relu_body(y2_ref, x_ref, ws_ref, stb_ref, bn_ref, o_ref,
                   *, imgs, count):
    totb = jnp.sum(stb_ref[...], axis=0)
    s2, t2 = _scale_shift_rows(totb[0:1], totb[1:2], count)
    bn = bn_ref[...]
    ss = bn[0:1]
    ts = bn[1:2]
    bs = bn[2:3]
    for i in range(imgs):
        sc = jnp.dot(x_ref[i].astype(jnp.bfloat16), ws_ref[...],
                     preferred_element_type=jnp.float32) + bs
        y2 = y2_ref[i].astype(jnp.float32) * s2 + t2
        o_ref[i] = jnp.maximum(y2 + sc * ss + ts, 0.0)


def kernel(x, w1, b1, w2, b2, ws, bs):
    N, cin, H, W = x.shape
    cout = w1.shape[-1]
    HW = H * W
    count = N * HW
    imgs = 8 if N % 8 == 0 else (2 if N % 2 == 0 else 1)
    imgs_c = 8 if N % 8 == 0 else (4 if N % 4 == 0 else 1)
    G = N // imgs
    Gc = N // imgs_c

    xh = jnp.transpose(x, (0, 2, 3, 1)).astype(jnp.float32).reshape(N, HW, cin)

    w1f = w1.reshape(9 * cin, cout).astype(jnp.bfloat16)
    w2f = w2.reshape(9 * cout, cout).astype(jnp.bfloat16)
    wsf = ws.astype(jnp.bfloat16)
    b1f = b1.reshape(1, cout)
    b2f = b2.reshape(1, cout)

    x_spec = pl.BlockSpec((imgs, HW, cin), lambda n: (n, 0, 0))
    xc_spec = pl.BlockSpec((imgs_c, HW, cin), lambda n: (n, 0, 0))
    row_spec = pl.BlockSpec((imgs, HW, cout), lambda n: (n, 0, 0))
    rowc_spec = pl.BlockSpec((imgs_c, HW, cout), lambda n: (n, 0, 0))
    st3_spec = pl.BlockSpec((1, 3, cout), lambda n: (n, 0, 0))
    st2_spec = pl.BlockSpec((1, 2, cout), lambda n: (n, 0, 0))
    g_spec = pl.BlockSpec((1, cin, cin), lambda n: (n, 0, 0))

    y1, sta, gram = pl.pallas_call(
        functools.partial(_conv1_body, H=H, W=W, imgs=imgs),
        grid=(G,),
        in_specs=[x_spec, _whole(w1f.shape), _whole((1, cout))],
        out_specs=(row_spec, st3_spec, g_spec),
        out_shape=(
            jax.ShapeDtypeStruct((N, HW, cout), jnp.bfloat16),
            jax.ShapeDtypeStruct((G, 3, cout), jnp.float32),
            jax.ShapeDtypeStruct((G, cin, cin), jnp.float32),
        ),
        scratch_shapes=[pltpu.VMEM((H + 2, W + 2, cin), jnp.bfloat16)],
        compiler_params=_cparams(),
        cost_estimate=pl.CostEstimate(
            flops=2 * count * (9 * cin + cin) * cout, transcendentals=0,
            bytes_accessed=4 * count * cin + 2 * count * cout),
    )(xh, w1f, b1f)

    y2, stb = pl.pallas_call(
        functools.partial(_conv2_body, H=H, W=W, imgs=imgs, count=count),
        grid=(G,),
        in_specs=[row_spec, _whole((G, 3, cout)), _whole(w2f.shape),
                  _whole((1, cout))],
        out_specs=(row_spec, st2_spec),
        out_shape=(jax.ShapeDtypeStruct((N, HW, cout), jnp.bfloat16),
                   jax.ShapeDtypeStruct((G, 2, cout), jnp.float32)),
        scratch_shapes=[pltpu.VMEM((H + 2, W + 2, cout), jnp.bfloat16)],
        compiler_params=_cparams(),
        cost_estimate=pl.CostEstimate(
            flops=2 * count * 9 * cout * cout, transcendentals=0,
            bytes_accessed=2 * 2 * count * cout),
    )(y1, sta, w2f, b2f)

    tota = jnp.sum(sta, axis=0)
    xs = tota[2][:cin]
    Gm = jnp.sum(gram, axis=0)
    wsf32 = wsf.astype(jnp.float32)
    mproj = xs @ wsf32
    ssum = mproj + count * bs
    qsum = (jnp.sum(wsf32 * (Gm @ wsf32), axis=0)
            + 2.0 * bs * mproj + count * bs * bs)
    ss_, ts_ = _scale_shift_rows(ssum[None], qsum[None], count)
    bnrows = jnp.concatenate([ss_, ts_, bs[None]], axis=0)

    out = pl.pallas_call(
        functools.partial(_add_relu_body, imgs=imgs_c, count=count),
        grid=(Gc,),
        in_specs=[rowc_spec, xc_spec, _whole(wsf.shape),
                  _whole((G, 2, cout)), _whole((3, cout))],
        out_specs=rowc_spec,
        out_shape=jax.ShapeDtypeStruct((N, HW, cout), jnp.float32),
        compiler_params=_cparams(),
        cost_estimate=pl.CostEstimate(
            flops=2 * count * cin * cout + 6 * count * cout, transcendentals=0,
            bytes_accessed=2 * count * cout + 4 * count * cin
                           + 4 * count * cout),
    )(y2, xh, wsf, stb, bnrows)

    return jnp.transpose(out.reshape(N, H, W, cout), (0, 3, 1, 2))
